# Initial kernel scaffold; baseline (speedup 1.0000x reference)
import jax
import jax.numpy as jnp
from jax.experimental import pallas as pl


def kernel(features, edge_index, W, b):
    raise NotImplementedError("write your pallas kernel here")


# speedup vs baseline: 2.9630x; 2.9630x over previous
import functools

import jax
import jax.numpy as jnp
from jax import lax
from jax.experimental import pallas as pl
from jax.experimental.pallas import tpu as pltpu
from jax.experimental.pallas import tpu_sc as plsc

N = 10000
E = 160000
D = 256
DH = 512

NC = 2
NS = 16
L = 16

_MESH = plsc.VectorSubcoreMesh(core_axis_name="c", subcore_axis_name="s")

EPW = E // (NC * NS)
_NVEC = (EPW + L - 1) // L
_IDXPAD = _NVEC * L


_DW = 128
_DCH = 40
_DNCH = EPW // _DCH


def _deg_body(src_hbm, dst_hbm, out_hbm, idx_sb, idx_db, sbuf, dbuf, table):
    cid = lax.axis_index("c")
    sid = lax.axis_index("s")

    zeros = jnp.zeros((L,), jnp.float32)
    lanes = lax.iota(jnp.int32, L)
    onehot = jnp.where(lanes == 0, 1.0, 0.0).astype(jnp.float32)

    def _zb(i, _):
        r = i // (_DW // L)
        g = i % (_DW // L)
        sbuf[r, pl.ds(g * L, L)] = zeros
        dbuf[r, pl.ds(g * L, L)] = zeros
        return 0

    lax.fori_loop(0, _DCH * (_DW // L), _zb, 0)

    def _zacc(k, _):
        ch = sid + k * NS

        @pl.when(ch < N // _DCH)
        def _():
            pltpu.sync_copy(sbuf, table.at[pl.ds(ch * _DCH, _DCH)])

        return 0

    lax.fori_loop(0, (N // _DCH + NS - 1) // NS, _zacc, 0)

    def _mk(r, _):
        sbuf[r, pl.ds(0, L)] = onehot
        dbuf[r, pl.ds(L, L)] = onehot
        return 0

    lax.fori_loop(0, _DCH, _mk, 0)
    plsc.subcore_barrier()

    def _chunk(i, _):
        base = cid * (E // NC) + sid * EPW + i * _DCH
        pltpu.sync_copy(src_hbm.at[pl.ds(base, _DCH)], idx_sb)
        pltpu.sync_copy(dst_hbm.at[pl.ds(base, _DCH)], idx_db)
        pltpu.sync_copy(sbuf, table.at[idx_sb], add=True)
        pltpu.sync_copy(dbuf, table.at[idx_db], add=True)
        return 0

    lax.fori_loop(0, _DNCH, _chunk, 0)
    plsc.subcore_barrier()

    @pl.when(sid < 10)
    def _():
        pltpu.sync_copy(table.at[pl.ds(sid * 1000, 1000)],
                        out_hbm.at[cid, pl.ds(sid * 1000, 1000)])


_deg_call = pl.kernel(
    _deg_body,
    out_type=jax.ShapeDtypeStruct((NC, N, _DW), jnp.float32),
    mesh=_MESH,
    scratch_types=[
        pltpu.VMEM((_DCH,), jnp.int32),
        pltpu.VMEM((_DCH,), jnp.int32),
        pltpu.VMEM((_DCH, _DW), jnp.float32),
        pltpu.VMEM((_DCH, _DW), jnp.float32),
        pltpu.VMEM_SHARED((N, _DW), jnp.float32),
    ],
)

_RB = 1000


def _scale_body(dgp_ref, feat_ref, out_ref):
    deg_o = dgp_ref[0, :, 0] + dgp_ref[1, :, 0]
    scale = lax.rsqrt(jnp.maximum(deg_o, 1.0))
    out_ref[0] = feat_ref[...] * scale[:, None]


_scale_call = pl.pallas_call(
    _scale_body,
    grid=(N // _RB, 2),
    in_specs=[
        pl.BlockSpec((NC, _RB, _DW), lambda i, j: (0, i, 0)),
        pl.BlockSpec((_RB, 128), lambda i, j: (i, j)),
    ],
    out_specs=pl.BlockSpec((1, _RB, 128), lambda i, j: (j, i, 0)),
    out_shape=jax.ShapeDtypeStruct((2, N, 128), jnp.float32),
)

EPT = E // NS
CH = 80
NCHUNK = EPT // CH
RPT = N // NS
_NZCH = (N // CH + NS - 1) // NS


def _agg_body(h0, h1, src_hbm, dst_hbm, out_hbm, idx_s, idx_d, rows, acc, sem):
    cid = lax.axis_index("c")
    sid = lax.axis_index("s")

    zeros = jnp.zeros((L,), jnp.float32)

    def _zrows(i, _):
        r = i // (128 // L)
        c = i % (128 // L)
        rows[r, pl.ds(c * L, L)] = zeros
        return 0

    lax.fori_loop(0, CH * (128 // L), _zrows, 0)

    def _zacc(k, _):
        ch = sid + k * NS

        @pl.when(ch < N // CH)
        def _():
            pltpu.sync_copy(rows, acc.at[pl.ds(ch * CH, CH)])

        return 0

    lax.fori_loop(0, _NZCH, _zacc, 0)
    plsc.subcore_barrier()

    def _run(table):
        def _chunk(i, _):
            base = sid * EPT + i * CH
            pltpu.sync_copy(src_hbm.at[pl.ds(base, CH)], idx_s)
            pltpu.sync_copy(dst_hbm.at[pl.ds(base, CH)], idx_d)
            pltpu.async_copy(table.at[idx_s], rows, sem).wait()
            pltpu.sync_copy(rows, acc.at[idx_d], add=True)
            return 0

        lax.fori_loop(0, NCHUNK, _chunk, 0)

    @pl.when(cid == 0)
    def _():
        _run(h0)

    @pl.when(cid == 1)
    def _():
        _run(h1)

    plsc.subcore_barrier()

    @pl.when(sid < 10)
    def _():
        pltpu.sync_copy(acc.at[pl.ds(sid * 1000, 1000)],
                        out_hbm.at[cid, pl.ds(sid * 1000, 1000)])


_agg_call = pl.kernel(
    _agg_body,
    out_type=jax.ShapeDtypeStruct((NC, N, 128), jnp.float32),
    mesh=_MESH,
    scratch_types=[
        pltpu.VMEM((CH,), jnp.int32),
        pltpu.VMEM((CH,), jnp.int32),
        pltpu.VMEM((CH, 128), jnp.float32),
        pltpu.VMEM_SHARED((N, 128), jnp.float32),
        pltpu.SemaphoreType.DMA,
    ],
)


def _mm_body(a0_ref, a1_ref, dgp_ref, w0_ref, w1_ref, b_ref, out_ref):
    deg_i = dgp_ref[0, :, L] + dgp_ref[1, :, L]
    scale = lax.rsqrt(jnp.maximum(deg_i, 1.0))
    x0 = a0_ref[...] * scale[:, None]
    x1 = a1_ref[...] * scale[:, None]
    y = (jnp.dot(x0, w0_ref[...], preferred_element_type=jnp.float32)
         + jnp.dot(x1, w1_ref[...], preferred_element_type=jnp.float32)
         + b_ref[...])
    out_ref[...] = jnp.maximum(y, 0.0)


_mm_call = pl.pallas_call(
    _mm_body,
    grid=(N // _RB,),
    in_specs=[
        pl.BlockSpec((_RB, 128), lambda i: (i, 0)),
        pl.BlockSpec((_RB, 128), lambda i: (i, 0)),
        pl.BlockSpec((NC, _RB, _DW), lambda i: (0, i, 0)),
        pl.BlockSpec((128, DH), lambda i: (0, 0)),
        pl.BlockSpec((128, DH), lambda i: (0, 0)),
        pl.BlockSpec((1, DH), lambda i: (0, 0)),
    ],
    out_specs=pl.BlockSpec((_RB, DH), lambda i: (i, 0)),
    out_shape=jax.ShapeDtypeStruct((N, DH), jnp.float32),
)


def kernel(features, edge_index, W, b):
    src = edge_index[0].astype(jnp.int32)
    dst = edge_index[1].astype(jnp.int32)
    dgp = _deg_call(src, dst)
    h_split = _scale_call(dgp, features)
    agg = _agg_call(h_split[0], h_split[1], src, dst)
    return _mm_call(agg[0], agg[1], dgp,
                    W[:128], W[128:], b.reshape(1, DH))


# speedup vs baseline: 3.3996x; 1.1473x over previous
import functools

import jax
import jax.numpy as jnp
from jax import lax
from jax.experimental import pallas as pl
from jax.experimental.pallas import tpu as pltpu
from jax.experimental.pallas import tpu_sc as plsc

N = 10000
E = 160000
D = 256
DH = 512

NC = 2
NS = 16
L = 16

ECH = 64
ECHUNKS = 2560
E_PAD = ECHUNKS * ECH
TRASH = N
ACC_R = 10112

_MESH = plsc.VectorSubcoreMesh(core_axis_name="c", subcore_axis_name="s")

EPW = E // (NC * NS)
_NVEC = (EPW + L - 1) // L
_IDXPAD = _NVEC * L


_DW = 128
_DCPT = ECHUNKS // NC // NS


def _deg_body(src_hbm, dst_hbm, out_hbm, islab_s, islab_d, sbuf, dbuf, table):
    cid = lax.axis_index("c")
    sid = lax.axis_index("s")

    zeros = jnp.zeros((L,), jnp.float32)
    lanes = lax.iota(jnp.int32, L)
    onehot = jnp.where(lanes == 0, 1.0, 0.0).astype(jnp.float32)

    def _zb(i, _):
        r = i // (_DW // L)
        g = i % (_DW // L)
        sbuf[r, pl.ds(g * L, L)] = zeros
        dbuf[r, pl.ds(g * L, L)] = zeros
        return 0

    lax.fori_loop(0, ECH * (_DW // L), _zb, 0)

    def _zacc(k, _):
        ch = sid + k * NS

        @pl.when(ch < ACC_R // ECH)
        def _():
            pltpu.sync_copy(sbuf, table.at[pl.ds(ch * ECH, ECH)])

        return 0

    lax.fori_loop(0, (ACC_R // ECH + NS - 1) // NS, _zacc, 0)

    start = cid * (ECHUNKS // NC) + sid * _DCPT
    pltpu.sync_copy(src_hbm.at[pl.ds(start, _DCPT)], islab_s)
    pltpu.sync_copy(dst_hbm.at[pl.ds(start, _DCPT)], islab_d)

    def _mk(r, _):
        sbuf[r, pl.ds(0, L)] = onehot
        dbuf[r, pl.ds(L, L)] = onehot
        return 0

    lax.fori_loop(0, ECH, _mk, 0)
    plsc.subcore_barrier()

    def _chunk(j, _):
        pltpu.sync_copy(sbuf, table.at[islab_s.at[j]], add=True)
        pltpu.sync_copy(dbuf, table.at[islab_d.at[j]], add=True)
        return 0

    lax.fori_loop(0, _DCPT, _chunk, 0)
    plsc.subcore_barrier()

    @pl.when(sid < 10)
    def _():
        pltpu.sync_copy(table.at[pl.ds(sid * 1000, 1000)],
                        out_hbm.at[cid, pl.ds(sid * 1000, 1000)])


_deg_call = pl.kernel(
    _deg_body,
    out_type=jax.ShapeDtypeStruct((NC, N, _DW), jnp.float32),
    mesh=_MESH,
    scratch_types=[
        pltpu.VMEM((_DCPT, ECH), jnp.int32),
        pltpu.VMEM((_DCPT, ECH), jnp.int32),
        pltpu.VMEM((ECH, _DW), jnp.float32),
        pltpu.VMEM((ECH, _DW), jnp.float32),
        pltpu.VMEM_SHARED((ACC_R, _DW), jnp.float32),
    ],
)

_RB = 1000


def _scale_body(dgp_ref, feat_ref, out_ref):
    deg_o = dgp_ref[0, :, 0] + dgp_ref[1, :, 0]
    scale = lax.rsqrt(jnp.maximum(deg_o, 1.0))
    out_ref[0] = feat_ref[...] * scale[:, None]


_scale_call = pl.pallas_call(
    _scale_body,
    grid=(N // _RB, 2),
    in_specs=[
        pl.BlockSpec((NC, _RB, _DW), lambda i, j: (0, i, 0)),
        pl.BlockSpec((_RB, 128), lambda i, j: (i, j)),
    ],
    out_specs=pl.BlockSpec((1, _RB, 128), lambda i, j: (j, i, 0)),
    out_shape=jax.ShapeDtypeStruct((2, N, 128), jnp.float32),
)

_ACPT = ECHUNKS // NS


def _agg_body(h0, h1, src_hbm, dst_hbm, out_hbm,
              islab_s, islab_d, buf0, buf1, acc, sem0, sem1):
    cid = lax.axis_index("c")
    sid = lax.axis_index("s")

    zeros = jnp.zeros((L,), jnp.float32)

    def _zrows(i, _):
        r = i // (128 // L)
        c = i % (128 // L)
        buf0[r, pl.ds(c * L, L)] = zeros
        return 0

    lax.fori_loop(0, ECH * (128 // L), _zrows, 0)

    def _zacc(k, _):
        ch = sid + k * NS

        @pl.when(ch < ACC_R // ECH)
        def _():
            pltpu.sync_copy(buf0, acc.at[pl.ds(ch * ECH, ECH)])

        return 0

    lax.fori_loop(0, (ACC_R // ECH + NS - 1) // NS, _zacc, 0)

    plsc.subcore_barrier()

    _HALF = _ACPT // 2

    def _run(table):
        for half in range(2):
            base = sid * _ACPT + half * _HALF
            pltpu.sync_copy(src_hbm.at[pl.ds(base, _HALF)], islab_s)
            pltpu.sync_copy(dst_hbm.at[pl.ds(base, _HALF)], islab_d)
            pltpu.async_copy(table.at[islab_s.at[0]], buf0, sem0)

            def _step(j, _):
                @pl.when(j % 2 == 0)
                def _():
                    pltpu.make_async_copy(
                        table.at[islab_s.at[j]], buf0, sem0).wait()

                    @pl.when(j + 1 < _HALF)
                    def _():
                        pltpu.async_copy(table.at[islab_s.at[j + 1]], buf1, sem1)

                    pltpu.sync_copy(buf0, acc.at[islab_d.at[j]], add=True)

                @pl.when(j % 2 == 1)
                def _():
                    pltpu.make_async_copy(
                        table.at[islab_s.at[j]], buf1, sem1).wait()

                    @pl.when(j + 1 < _HALF)
                    def _():
                        pltpu.async_copy(table.at[islab_s.at[j + 1]], buf0, sem0)

                    pltpu.sync_copy(buf1, acc.at[islab_d.at[j]], add=True)

                return 0

            lax.fori_loop(0, _HALF, _step, 0)

    @pl.when(cid == 0)
    def _():
        _run(h0)

    @pl.when(cid == 1)
    def _():
        _run(h1)

    plsc.subcore_barrier()

    @pl.when(sid < 10)
    def _():
        pltpu.sync_copy(acc.at[pl.ds(sid * 1000, 1000)],
                        out_hbm.at[cid, pl.ds(sid * 1000, 1000)])


_agg_call = pl.kernel(
    _agg_body,
    out_type=jax.ShapeDtypeStruct((NC, N, 128), jnp.float32),
    mesh=_MESH,
    scratch_types=[
        pltpu.VMEM((_ACPT // 2, ECH), jnp.int32),
        pltpu.VMEM((_ACPT // 2, ECH), jnp.int32),
        pltpu.VMEM((ECH, 128), jnp.float32),
        pltpu.VMEM((ECH, 128), jnp.float32),
        pltpu.VMEM_SHARED((ACC_R, 128), jnp.float32),
        pltpu.SemaphoreType.DMA,
        pltpu.SemaphoreType.DMA,
    ],
)


def _mm_body(a0_ref, a1_ref, dgp_ref, w0_ref, w1_ref, b_ref, out_ref):
    deg_i = dgp_ref[0, :, L] + dgp_ref[1, :, L]
    scale = lax.rsqrt(jnp.maximum(deg_i, 1.0))
    x0 = a0_ref[...] * scale[:, None]
    x1 = a1_ref[...] * scale[:, None]
    y = (jnp.dot(x0, w0_ref[...], preferred_element_type=jnp.float32)
         + jnp.dot(x1, w1_ref[...], preferred_element_type=jnp.float32)
         + b_ref[...])
    out_ref[...] = jnp.maximum(y, 0.0)


_mm_call = pl.pallas_call(
    _mm_body,
    grid=(N // _RB,),
    in_specs=[
        pl.BlockSpec((_RB, 128), lambda i: (i, 0)),
        pl.BlockSpec((_RB, 128), lambda i: (i, 0)),
        pl.BlockSpec((NC, _RB, _DW), lambda i: (0, i, 0)),
        pl.BlockSpec((128, DH), lambda i: (0, 0)),
        pl.BlockSpec((128, DH), lambda i: (0, 0)),
        pl.BlockSpec((1, DH), lambda i: (0, 0)),
    ],
    out_specs=pl.BlockSpec((_RB, DH), lambda i: (i, 0)),
    out_shape=jax.ShapeDtypeStruct((N, DH), jnp.float32),
)


def kernel(features, edge_index, W, b):
    src = edge_index[0].astype(jnp.int32)
    dst = edge_index[1].astype(jnp.int32)
    pad = E_PAD - E
    trash = jnp.full((pad,), TRASH, jnp.int32)
    src_deg = jnp.concatenate([src, trash]).reshape(ECHUNKS, ECH)
    dst_pad = jnp.concatenate([dst, trash]).reshape(ECHUNKS, ECH)
    src_agg = jnp.concatenate(
        [src, jnp.zeros((pad,), jnp.int32)]).reshape(ECHUNKS, ECH)
    dgp = _deg_call(src_deg, dst_pad)
    h_split = _scale_call(dgp, features)
    agg = _agg_call(h_split[0], h_split[1], src_agg, dst_pad)
    return _mm_call(agg[0], agg[1], dgp,
                    W[:128], W[128:], b.reshape(1, DH))


# speedup vs baseline: 3.8826x; 1.1421x over previous
import functools

import jax
import jax.numpy as jnp
from jax import lax
from jax.experimental import pallas as pl
from jax.experimental.pallas import tpu as pltpu
from jax.experimental.pallas import tpu_sc as plsc

N = 10000
E = 160000
D = 256
DH = 512

NC = 2
NS = 16
L = 16

ECH = 64
ECHUNKS = 2560
E_PAD = ECHUNKS * ECH
TRASH = N
ACC_R = 10112

_MESH = plsc.VectorSubcoreMesh(core_axis_name="c", subcore_axis_name="s")

EPW = E // (NC * NS)
_NVEC = (EPW + L - 1) // L
_IDXPAD = _NVEC * L


_DW = 128
_DCPT = ECHUNKS // NC // NS


def _deg_body(src_hbm, dst_hbm, out_hbm, islab_s, islab_d, sbuf, dbuf, table):
    cid = lax.axis_index("c")
    sid = lax.axis_index("s")

    zeros = jnp.zeros((L,), jnp.float32)
    lanes = lax.iota(jnp.int32, L)
    onehot = jnp.where(lanes == 0, 1.0, 0.0).astype(jnp.float32)

    def _zb(i, _):
        r = i // (_DW // L)
        g = i % (_DW // L)
        sbuf[r, pl.ds(g * L, L)] = zeros
        dbuf[r, pl.ds(g * L, L)] = zeros
        return 0

    lax.fori_loop(0, ECH * (_DW // L), _zb, 0)

    def _zacc(k, _):
        ch = sid + k * NS

        @pl.when(ch < ACC_R // ECH)
        def _():
            pltpu.sync_copy(sbuf, table.at[pl.ds(ch * ECH, ECH)])

        return 0

    lax.fori_loop(0, (ACC_R // ECH + NS - 1) // NS, _zacc, 0)

    start = cid * (ECHUNKS // NC) + sid * _DCPT
    pltpu.sync_copy(src_hbm.at[pl.ds(start, _DCPT)], islab_s)
    pltpu.sync_copy(dst_hbm.at[pl.ds(start, _DCPT)], islab_d)

    def _mk(r, _):
        sbuf[r, pl.ds(0, L)] = onehot
        dbuf[r, pl.ds(L, L)] = onehot
        return 0

    lax.fori_loop(0, ECH, _mk, 0)
    plsc.subcore_barrier()

    def _chunk(j, _):
        pltpu.sync_copy(sbuf, table.at[islab_s.at[j]], add=True)
        pltpu.sync_copy(dbuf, table.at[islab_d.at[j]], add=True)
        return 0

    lax.fori_loop(0, _DCPT, _chunk, 0)
    plsc.subcore_barrier()

    @pl.when(sid < 10)
    def _():
        pltpu.sync_copy(table.at[pl.ds(sid * 1000, 1000)],
                        out_hbm.at[cid, pl.ds(sid * 1000, 1000)])


_deg_call = pl.kernel(
    _deg_body,
    out_type=jax.ShapeDtypeStruct((NC, N, _DW), jnp.float32),
    mesh=_MESH,
    scratch_types=[
        pltpu.VMEM((_DCPT, ECH), jnp.int32),
        pltpu.VMEM((_DCPT, ECH), jnp.int32),
        pltpu.VMEM((ECH, _DW), jnp.float32),
        pltpu.VMEM((ECH, _DW), jnp.float32),
        pltpu.VMEM_SHARED((ACC_R, _DW), jnp.float32),
    ],
)

_RB = 1000


def _scale_body(dgp_ref, feat_ref, out_ref):
    deg_o = dgp_ref[0, :, 0] + dgp_ref[1, :, 0]
    scale = lax.rsqrt(jnp.maximum(deg_o, 1.0))
    out_ref[0] = feat_ref[...] * scale[:, None]


_scale_call = pl.pallas_call(
    _scale_body,
    grid=(N // _RB, 2),
    in_specs=[
        pl.BlockSpec((NC, _RB, _DW), lambda i, j: (0, i, 0)),
        pl.BlockSpec((_RB, 128), lambda i, j: (i, j)),
    ],
    out_specs=pl.BlockSpec((1, _RB, 128), lambda i, j: (j, i, 0)),
    out_shape=jax.ShapeDtypeStruct((2, N, 128), jnp.float32),
)

_ACPT = ECHUNKS // NS


def _agg_body(h0, h1, src_hbm, dst_hbm, out_hbm,
              islab_s, islab_d, buf0, buf1, buf2, acc,
              gsem0, gsem1, gsem2, ssem0, ssem1, ssem2):
    cid = lax.axis_index("c")
    sid = lax.axis_index("s")

    zeros = jnp.zeros((L,), jnp.float32)

    def _zrows(i, _):
        r = i // (128 // L)
        c = i % (128 // L)
        buf0[r, pl.ds(c * L, L)] = zeros
        return 0

    lax.fori_loop(0, ECH * (128 // L), _zrows, 0)

    def _zacc(k, _):
        ch = sid + k * NS

        @pl.when(ch < ACC_R // ECH)
        def _():
            pltpu.sync_copy(buf0, acc.at[pl.ds(ch * ECH, ECH)])

        return 0

    lax.fori_loop(0, (ACC_R // ECH + NS - 1) // NS, _zacc, 0)

    plsc.subcore_barrier()

    _HALF = _ACPT // 2

    def _run(table):
        bufs = (buf0, buf1, buf2)
        gsems = (gsem0, gsem1, gsem2)
        ssems = (ssem0, ssem1, ssem2)

        for half in range(2):
            base = sid * _ACPT + half * _HALF
            pltpu.sync_copy(src_hbm.at[pl.ds(base, _HALF)], islab_s)
            pltpu.sync_copy(dst_hbm.at[pl.ds(base, _HALF)], islab_d)
            pltpu.async_copy(table.at[islab_s.at[0]], buf0, gsem0)
            pltpu.async_copy(table.at[islab_s.at[1]], buf1, gsem1)

            def _step(j, _):
                for r in range(3):
                    @pl.when(j % 3 == r)
                    def _(r=r):
                        b, gs, ss = bufs[r], gsems[r], ssems[r]
                        nb, ns = bufs[(r + 2) % 3], ssems[(r + 2) % 3]
                        pltpu.make_async_copy(
                            table.at[islab_s.at[j]], b, gs).wait()
                        pltpu.async_copy(
                            b, acc.at[islab_d.at[j]], ss, add=True)

                        @pl.when(j + 2 < _HALF)
                        def _():
                            @pl.when(j >= 1)
                            def _():
                                pltpu.make_async_copy(
                                    nb, acc.at[islab_d.at[j]], ns).wait()

                            pltpu.async_copy(
                                table.at[islab_s.at[j + 2]],
                                nb, gsems[(r + 2) % 3])

                return 0

            lax.fori_loop(0, _HALF, _step, 0)
            for r in range(3):
                pltpu.make_async_copy(
                    bufs[r], acc.at[islab_d.at[0]], ssems[r]).wait()

    @pl.when(cid == 0)
    def _():
        _run(h0)

    @pl.when(cid == 1)
    def _():
        _run(h1)

    plsc.subcore_barrier()

    @pl.when(sid < 10)
    def _():
        pltpu.sync_copy(acc.at[pl.ds(sid * 1000, 1000)],
                        out_hbm.at[cid, pl.ds(sid * 1000, 1000)])


_agg_call = pl.kernel(
    _agg_body,
    out_type=jax.ShapeDtypeStruct((NC, N, 128), jnp.float32),
    mesh=_MESH,
    scratch_types=[
        pltpu.VMEM((_ACPT // 2, ECH), jnp.int32),
        pltpu.VMEM((_ACPT // 2, ECH), jnp.int32),
        pltpu.VMEM((ECH, 128), jnp.float32),
        pltpu.VMEM((ECH, 128), jnp.float32),
        pltpu.VMEM((ECH, 128), jnp.float32),
        pltpu.VMEM_SHARED((ACC_R, 128), jnp.float32),
        pltpu.SemaphoreType.DMA,
        pltpu.SemaphoreType.DMA,
        pltpu.SemaphoreType.DMA,
        pltpu.SemaphoreType.DMA,
        pltpu.SemaphoreType.DMA,
        pltpu.SemaphoreType.DMA,
    ],
)


def _mm_body(a0_ref, a1_ref, dgp_ref, w0_ref, w1_ref, b_ref, out_ref):
    deg_i = dgp_ref[0, :, L] + dgp_ref[1, :, L]
    scale = lax.rsqrt(jnp.maximum(deg_i, 1.0))
    x0 = a0_ref[...] * scale[:, None]
    x1 = a1_ref[...] * scale[:, None]
    y = (jnp.dot(x0, w0_ref[...], preferred_element_type=jnp.float32)
         + jnp.dot(x1, w1_ref[...], preferred_element_type=jnp.float32)
         + b_ref[...])
    out_ref[...] = jnp.maximum(y, 0.0)


_mm_call = pl.pallas_call(
    _mm_body,
    grid=(N // _RB,),
    in_specs=[
        pl.BlockSpec((_RB, 128), lambda i: (i, 0)),
        pl.BlockSpec((_RB, 128), lambda i: (i, 0)),
        pl.BlockSpec((NC, _RB, _DW), lambda i: (0, i, 0)),
        pl.BlockSpec((128, DH), lambda i: (0, 0)),
        pl.BlockSpec((128, DH), lambda i: (0, 0)),
        pl.BlockSpec((1, DH), lambda i: (0, 0)),
    ],
    out_specs=pl.BlockSpec((_RB, DH), lambda i: (i, 0)),
    out_shape=jax.ShapeDtypeStruct((N, DH), jnp.float32),
)


def kernel(features, edge_index, W, b):
    src = edge_index[0].astype(jnp.int32)
    dst = edge_index[1].astype(jnp.int32)
    pad = E_PAD - E
    trash = jnp.full((pad,), TRASH, jnp.int32)
    src_deg = jnp.concatenate([src, trash]).reshape(ECHUNKS, ECH)
    dst_pad = jnp.concatenate([dst, trash]).reshape(ECHUNKS, ECH)
    src_agg = jnp.concatenate(
        [src, jnp.zeros((pad,), jnp.int32)]).reshape(ECHUNKS, ECH)
    dgp = _deg_call(src_deg, dst_pad)
    h_split = _scale_call(dgp, features)
    agg = _agg_call(h_split[0], h_split[1], src_agg, dst_pad)
    return _mm_call(agg[0], agg[1], dgp,
                    W[:128], W[128:], b.reshape(1, DH))


# speedup vs baseline: 4.1971x; 1.0810x over previous
import functools

import jax
import jax.numpy as jnp
from jax import lax
from jax.experimental import pallas as pl
from jax.experimental.pallas import tpu as pltpu
from jax.experimental.pallas import tpu_sc as plsc

N = 10000
E = 160000
D = 256
DH = 512

NC = 2
NS = 16
L = 16

ECH = 80
ECHUNKS = 2048
E_PAD = ECHUNKS * ECH
TRASH = N
ACC_R = 10080

_MESH = plsc.VectorSubcoreMesh(core_axis_name="c", subcore_axis_name="s")

EPW = E // (NC * NS)
_NVEC = (EPW + L - 1) // L
_IDXPAD = _NVEC * L


_DW = 128
_DCPT = ECHUNKS // NC // NS


def _deg_body(src_hbm, dst_hbm, out_hbm, islab_s, islab_d, sbuf, dbuf, table,
              asem):
    cid = lax.axis_index("c")
    sid = lax.axis_index("s")

    zeros = jnp.zeros((L,), jnp.float32)
    lanes = lax.iota(jnp.int32, L)
    onehot = jnp.where(lanes == 0, 1.0, 0.0).astype(jnp.float32)

    def _zb(i, _):
        r = i // (_DW // L)
        g = i % (_DW // L)
        sbuf[r, pl.ds(g * L, L)] = zeros
        dbuf[r, pl.ds(g * L, L)] = zeros
        return 0

    lax.fori_loop(0, ECH * (_DW // L), _zb, 0)

    def _zacc(k, _):
        ch = sid + k * NS

        @pl.when(ch < ACC_R // ECH)
        def _():
            pltpu.sync_copy(sbuf, table.at[pl.ds(ch * ECH, ECH)])

        return 0

    lax.fori_loop(0, (ACC_R // ECH + NS - 1) // NS, _zacc, 0)

    start = cid * (ECHUNKS // NC) + sid * _DCPT
    pltpu.sync_copy(src_hbm.at[pl.ds(start, _DCPT)], islab_s)
    pltpu.sync_copy(dst_hbm.at[pl.ds(start, _DCPT)], islab_d)

    def _mk(r, _):
        sbuf[r, pl.ds(0, L)] = onehot
        dbuf[r, pl.ds(L, L)] = onehot
        return 0

    lax.fori_loop(0, ECH, _mk, 0)
    plsc.subcore_barrier()

    def _chunk(j, _):
        pltpu.async_copy(sbuf, table.at[islab_s.at[j]], asem, add=True)
        pltpu.async_copy(dbuf, table.at[islab_d.at[j]], asem, add=True)

        @pl.when(j >= 4)
        def _():
            pltpu.make_async_copy(sbuf, table.at[islab_s.at[0]], asem).wait()
            pltpu.make_async_copy(dbuf, table.at[islab_d.at[0]], asem).wait()

        return 0

    lax.fori_loop(0, _DCPT, _chunk, 0)

    def _drain(j, _):
        pltpu.make_async_copy(sbuf, table.at[islab_s.at[0]], asem).wait()
        pltpu.make_async_copy(dbuf, table.at[islab_d.at[0]], asem).wait()
        return 0

    lax.fori_loop(0, 4, _drain, 0)
    plsc.subcore_barrier()

    @pl.when(sid < 10)
    def _():
        pltpu.sync_copy(table.at[pl.ds(sid * 1000, 1000)],
                        out_hbm.at[cid, pl.ds(sid * 1000, 1000)])


_deg_call = pl.kernel(
    _deg_body,
    out_type=jax.ShapeDtypeStruct((NC, N, _DW), jnp.float32),
    mesh=_MESH,
    scratch_types=[
        pltpu.VMEM((_DCPT, ECH), jnp.int32),
        pltpu.VMEM((_DCPT, ECH), jnp.int32),
        pltpu.VMEM((ECH, _DW), jnp.float32),
        pltpu.VMEM((ECH, _DW), jnp.float32),
        pltpu.VMEM_SHARED((ACC_R, _DW), jnp.float32),
        pltpu.SemaphoreType.DMA,
    ],
)

_RB = 1000


def _scale_body(dgp_ref, feat_ref, out_ref):
    deg_o = dgp_ref[0, :, 0] + dgp_ref[1, :, 0]
    scale = lax.rsqrt(jnp.maximum(deg_o, 1.0))
    out_ref[0] = feat_ref[...] * scale[:, None]


_scale_call = pl.pallas_call(
    _scale_body,
    grid=(N // _RB, 2),
    in_specs=[
        pl.BlockSpec((NC, _RB, _DW), lambda i, j: (0, i, 0)),
        pl.BlockSpec((_RB, 128), lambda i, j: (i, j)),
    ],
    out_specs=pl.BlockSpec((1, _RB, 128), lambda i, j: (j, i, 0)),
    out_shape=jax.ShapeDtypeStruct((2, N, 128), jnp.float32),
)

_ACPT = ECHUNKS // NS


def _agg_body(h0, h1, src_hbm, dst_hbm, out_hbm,
              islab_s, islab_d, buf0, buf1, buf2, acc,
              gsem0, gsem1, gsem2, ssem0, ssem1, ssem2):
    cid = lax.axis_index("c")
    sid = lax.axis_index("s")

    zeros = jnp.zeros((L,), jnp.float32)

    def _zrows(i, _):
        r = i // (128 // L)
        c = i % (128 // L)
        buf0[r, pl.ds(c * L, L)] = zeros
        return 0

    lax.fori_loop(0, ECH * (128 // L), _zrows, 0)

    def _zacc(k, _):
        ch = sid + k * NS

        @pl.when(ch < ACC_R // ECH)
        def _():
            pltpu.sync_copy(buf0, acc.at[pl.ds(ch * ECH, ECH)])

        return 0

    lax.fori_loop(0, (ACC_R // ECH + NS - 1) // NS, _zacc, 0)

    plsc.subcore_barrier()

    _HALF = _ACPT // 2

    def _run(table):
        bufs = (buf0, buf1, buf2)
        gsems = (gsem0, gsem1, gsem2)
        ssems = (ssem0, ssem1, ssem2)

        for half in range(2):
            base = sid * _ACPT + half * _HALF
            pltpu.sync_copy(src_hbm.at[pl.ds(base, _HALF)], islab_s)
            pltpu.sync_copy(dst_hbm.at[pl.ds(base, _HALF)], islab_d)
            pltpu.async_copy(table.at[islab_s.at[0]], buf0, gsem0)
            pltpu.async_copy(table.at[islab_s.at[1]], buf1, gsem1)

            def _step(j, _):
                for r in range(3):
                    @pl.when(j % 3 == r)
                    def _(r=r):
                        b, gs, ss = bufs[r], gsems[r], ssems[r]
                        nb, ns = bufs[(r + 2) % 3], ssems[(r + 2) % 3]
                        pltpu.make_async_copy(
                            table.at[islab_s.at[j]], b, gs).wait()
                        pltpu.async_copy(
                            b, acc.at[islab_d.at[j]], ss, add=True)

                        @pl.when(j + 2 < _HALF)
                        def _():
                            @pl.when(j >= 1)
                            def _():
                                pltpu.make_async_copy(
                                    nb, acc.at[islab_d.at[j]], ns).wait()

                            pltpu.async_copy(
                                table.at[islab_s.at[j + 2]],
                                nb, gsems[(r + 2) % 3])

                return 0

            lax.fori_loop(0, _HALF, _step, 0)
            for r in range(3):
                pltpu.make_async_copy(
                    bufs[r], acc.at[islab_d.at[0]], ssems[r]).wait()

    @pl.when(cid == 0)
    def _():
        _run(h0)

    @pl.when(cid == 1)
    def _():
        _run(h1)

    plsc.subcore_barrier()

    @pl.when(sid < 10)
    def _():
        pltpu.sync_copy(acc.at[pl.ds(sid * 1000, 1000)],
                        out_hbm.at[cid, pl.ds(sid * 1000, 1000)])


_agg_call = pl.kernel(
    _agg_body,
    out_type=jax.ShapeDtypeStruct((NC, N, 128), jnp.float32),
    mesh=_MESH,
    scratch_types=[
        pltpu.VMEM((_ACPT // 2, ECH), jnp.int32),
        pltpu.VMEM((_ACPT // 2, ECH), jnp.int32),
        pltpu.VMEM((ECH, 128), jnp.float32),
        pltpu.VMEM((ECH, 128), jnp.float32),
        pltpu.VMEM((ECH, 128), jnp.float32),
        pltpu.VMEM_SHARED((ACC_R, 128), jnp.float32),
        pltpu.SemaphoreType.DMA,
        pltpu.SemaphoreType.DMA,
        pltpu.SemaphoreType.DMA,
        pltpu.SemaphoreType.DMA,
        pltpu.SemaphoreType.DMA,
        pltpu.SemaphoreType.DMA,
    ],
)


def _mm_body(a0_ref, a1_ref, dgp_ref, w0_ref, w1_ref, b_ref, out_ref):
    deg_i = dgp_ref[0, :, L] + dgp_ref[1, :, L]
    scale = lax.rsqrt(jnp.maximum(deg_i, 1.0))
    x0 = a0_ref[...] * scale[:, None]
    x1 = a1_ref[...] * scale[:, None]
    y = (jnp.dot(x0, w0_ref[...], preferred_element_type=jnp.float32)
         + jnp.dot(x1, w1_ref[...], preferred_element_type=jnp.float32)
         + b_ref[...])
    out_ref[...] = jnp.maximum(y, 0.0)


_mm_call = pl.pallas_call(
    _mm_body,
    grid=(N // _RB,),
    in_specs=[
        pl.BlockSpec((_RB, 128), lambda i: (i, 0)),
        pl.BlockSpec((_RB, 128), lambda i: (i, 0)),
        pl.BlockSpec((NC, _RB, _DW), lambda i: (0, i, 0)),
        pl.BlockSpec((128, DH), lambda i: (0, 0)),
        pl.BlockSpec((128, DH), lambda i: (0, 0)),
        pl.BlockSpec((1, DH), lambda i: (0, 0)),
    ],
    out_specs=pl.BlockSpec((_RB, DH), lambda i: (i, 0)),
    out_shape=jax.ShapeDtypeStruct((N, DH), jnp.float32),
)


def kernel(features, edge_index, W, b):
    src = edge_index[0].astype(jnp.int32)
    dst = edge_index[1].astype(jnp.int32)
    pad = E_PAD - E
    trash = jnp.full((pad,), TRASH, jnp.int32)
    src_deg = jnp.concatenate([src, trash]).reshape(ECHUNKS, ECH)
    dst_pad = jnp.concatenate([dst, trash]).reshape(ECHUNKS, ECH)
    src_agg = jnp.concatenate(
        [src, jnp.zeros((pad,), jnp.int32)]).reshape(ECHUNKS, ECH)
    dgp = _deg_call(src_deg, dst_pad)
    h_split = _scale_call(dgp, features)
    agg = _agg_call(h_split[0], h_split[1], src_agg, dst_pad)
    return _mm_call(agg[0], agg[1], dgp,
                    W[:128], W[128:], b.reshape(1, DH))
